# Initial kernel scaffold; baseline (speedup 1.0000x reference)
import jax
import jax.numpy as jnp
from jax.experimental import pallas as pl


def kernel(enc1_w, enc1_b, enc2_w, enc2_b, enc3_w, enc3_b, z_w, z_b, dec1_w, dec1_b, dec2_w, dec2_b, dec3_w, dec3_b, xbar_w, xbar_b, gnn1_w, gnn2_w, gnn3_w, gnn4_w, gnn5_w, cluster_layer, disc_weight, x, adj, perm):
    raise NotImplementedError("write your pallas kernel here")


# speedup vs baseline: 1.1227x; 1.1227x over previous
import functools

import jax
import jax.numpy as jnp
from jax.experimental import pallas as pl
from jax.experimental.pallas import tpu as pltpu

_SIGMA = 0.5


def _spec2(shape):
    return pl.BlockSpec(shape, lambda i: (0, 0))


def _fused_kernel(
    x_ref, adj_hbm_ref, permf_ref,
    e1w_ref, e1b_ref, e2w_ref, e2b_ref, e3w_ref, e3b_ref, zw_ref, zb_ref,
    d1w_ref, d1b_ref, d2w_ref, d2b_ref, d3w_ref, d3b_ref, xbw_ref, xbb_ref,
    g1w_ref, g2w_ref, g3w_ref, g4w_ref, g5w_ref, clu_ref,
    h_ref, pred_ref, q_ref, z_ref, outh_ref, negh_ref, xbar_ref, summ_ref,
    adj_vmem, adj_sem,
    *, v,
):
    f32 = jnp.float32
    bf16 = jnp.bfloat16
    s = _SIGMA
    n = x_ref.shape[0]

    adj_cp = pltpu.make_async_copy(adj_hbm_ref, adj_vmem, adj_sem)
    adj_cp.start()

    def lin(xv, w_ref, b_ref, relu):
        y = jnp.dot(xv.astype(bf16), w_ref[...], preferred_element_type=f32)
        y = y + b_ref[...]
        return jnp.maximum(y, 0.0) if relu else y

    def mm(xv, w_ref):
        return jnp.dot(xv.astype(bf16), w_ref[...], preferred_element_type=f32)

    xb = x_ref[...]

    t1 = lin(xb, e1w_ref, e1b_ref, True)
    t2 = lin(t1, e2w_ref, e2b_ref, True)
    t3 = lin(t2, e3w_ref, e3b_ref, True)

    z = lin(t3, zw_ref, zb_ref, False)
    d1 = lin(z, d1w_ref, d1b_ref, True)
    d2 = lin(d1, d2w_ref, d2b_ref, True)
    d3 = lin(d2, d3w_ref, d3b_ref, True)
    x_bar = lin(d3, xbw_ref, xbb_ref, False)

    lane = jax.lax.broadcasted_iota(jnp.int32, (n, n), 1).astype(f32)
    pmat = (lane == permf_ref[...]).astype(bf16)
    tcat = jnp.concatenate(
        [t1.astype(bf16), t2.astype(bf16), t3.astype(bf16)], axis=1)
    ncat = jnp.dot(pmat, tcat, preferred_element_type=f32)
    c1 = t1.shape[1]
    c2 = c1 + t2.shape[1]
    nt1 = ncat[:, :c1]
    nt2 = ncat[:, c1:c2]
    nt3 = ncat[:, c2:]

    adj_cp.wait()
    adj = adj_vmem[...]

    def agg(sup, relu):
        y = jnp.dot(adj, sup.astype(bf16), preferred_element_type=f32)
        return jnp.maximum(y, 0.0) if relu else y

    h1 = agg(mm(xb, g1w_ref), True)

    e2_ = g2w_ref.shape[1]
    mix2 = jnp.concatenate([(1.0 - s) * h1 + s * t1,
                            (1.0 - s) * h1 + s * nt1], axis=0)
    sup2 = mm(mix2, g2w_ref)
    a2 = agg(jnp.concatenate([sup2[:n], sup2[n:]], axis=1), True)
    h2, nh2 = a2[:, :e2_], a2[:, e2_:]

    e3_ = g3w_ref.shape[1]
    mix3 = jnp.concatenate([(1.0 - s) * h2 + s * t2,
                            (1.0 - s) * nh2 + s * nt2], axis=0)
    sup3 = mm(mix3, g3w_ref)
    a3 = agg(jnp.concatenate([sup3[:n], sup3[n:]], axis=1), True)
    h3, nh3 = a3[:, :e3_], a3[:, e3_:]

    out_h = (1.0 - s) * h3 + s * t3
    neg_h = (1.0 - s) * nh3 + s * nt3

    h4 = agg(mm(out_h, g4w_ref), True)
    h5 = agg(mm((1.0 - s) * h4 + s * z, g5w_ref), False)

    m = jnp.max(h5, axis=1, keepdims=True)
    e = jnp.exp(h5 - m)
    pred = e * pl.reciprocal(jnp.sum(e, axis=1, keepdims=True), approx=True)

    mean = jnp.mean(out_h, axis=0, keepdims=True)
    en = jnp.exp(-jnp.abs(mean))
    r = pl.reciprocal(1.0 + en, approx=True)
    summ_ref[...] = jnp.where(mean >= 0.0, r, en * r)

    c = clu_ref[...]
    z2 = jnp.sum(z * z, axis=1, keepdims=True)
    c2 = jnp.sum(c * c, axis=1, keepdims=True).T
    zc = jnp.dot(z, c.T, preferred_element_type=f32)
    dist = jnp.maximum(z2 + c2 - 2.0 * zc, 0.0)
    q = pl.reciprocal(1.0 + dist * (1.0 / v), approx=True)
    if v != 1.0:
        q = q ** ((v + 1.0) / 2.0)
    q = q * pl.reciprocal(jnp.sum(q, axis=1, keepdims=True), approx=True)

    h_ref[...] = h5
    pred_ref[...] = pred
    q_ref[...] = q
    z_ref[...] = z
    outh_ref[...] = out_h
    negh_ref[...] = neg_h
    xbar_ref[...] = x_bar


def kernel(enc1_w, enc1_b, enc2_w, enc2_b, enc3_w, enc3_b, z_w, z_b,
           dec1_w, dec1_b, dec2_w, dec2_b, dec3_w, dec3_b, xbar_w, xbar_b,
           gnn1_w, gnn2_w, gnn3_w, gnn4_w, gnn5_w, cluster_layer, disc_weight,
           x, adj, perm):
    n, n_input = x.shape
    n_enc_3 = enc3_w.shape[1]
    n_z = z_w.shape[1]
    n_clusters = gnn5_w.shape[1]
    v = 1.0

    f32 = jnp.float32
    bf16 = jnp.bfloat16

    def b2(b):
        return b.reshape(1, -1).astype(f32)

    def w16(w):
        return w.astype(bf16)

    operands = [
        x.astype(bf16), adj.astype(bf16),
        perm.astype(f32).reshape(n, 1),
        w16(enc1_w), b2(enc1_b), w16(enc2_w), b2(enc2_b),
        w16(enc3_w), b2(enc3_b), w16(z_w), b2(z_b),
        w16(dec1_w), b2(dec1_b), w16(dec2_w), b2(dec2_b),
        w16(dec3_w), b2(dec3_b), w16(xbar_w), b2(xbar_b),
        w16(gnn1_w), w16(gnn2_w), w16(gnn3_w), w16(gnn4_w), w16(gnn5_w),
        cluster_layer.astype(f32),
    ]

    in_specs = []
    for idx, op in enumerate(operands):
        if idx == 1:
            in_specs.append(pl.BlockSpec(memory_space=pl.ANY))
        else:
            in_specs.append(_spec2(op.shape))

    out_shapes = (
        jax.ShapeDtypeStruct((n, n_clusters), f32),
        jax.ShapeDtypeStruct((n, n_clusters), f32),
        jax.ShapeDtypeStruct((n, n_clusters), f32),
        jax.ShapeDtypeStruct((n, n_z), f32),
        jax.ShapeDtypeStruct((n, n_enc_3), f32),
        jax.ShapeDtypeStruct((n, n_enc_3), f32),
        jax.ShapeDtypeStruct((n, n_input), f32),
        jax.ShapeDtypeStruct((1, n_enc_3), f32),
    )
    out_specs = [_spec2(sh.shape) for sh in out_shapes]

    outs = pl.pallas_call(
        functools.partial(_fused_kernel, v=v),
        grid=(1,),
        in_specs=in_specs,
        out_specs=out_specs,
        out_shape=out_shapes,
        scratch_shapes=[
            pltpu.VMEM((n, n), bf16),
            pltpu.SemaphoreType.DMA(()),
        ],
        compiler_params=pltpu.CompilerParams(
            dimension_semantics=("arbitrary",),
            vmem_limit_bytes=64 * 1024 * 1024,
        ),
    )(*operands)

    h, pred, q, z, out_h, neg_h, x_bar, summary = outs
    return h, out_h, neg_h, summary.reshape(-1), x_bar, q, pred, z


# speedup vs baseline: 1.7538x; 1.5621x over previous
import functools

import jax
import jax.numpy as jnp
from jax.experimental import pallas as pl
from jax.experimental.pallas import tpu as pltpu

_SIGMA = 0.5


def _spec2(shape):
    return pl.BlockSpec(shape, lambda i: (0, 0))


def _fused_kernel(
    x_ref, permi_ref,
    e1b_ref, e2b_ref, e3b_ref, zb_ref,
    d1b_ref, d2b_ref, d3b_ref, xbb_ref,
    g5w_ref, clu_ref,
    adj_hbm, e1w_hbm, e2w_hbm, e3w_hbm, zw_hbm,
    d1w_hbm, d2w_hbm, d3w_hbm, xbw_hbm,
    g1w_hbm, g2w_hbm, g3w_hbm, g4w_hbm,
    h_ref, pred_ref, q_ref, z_ref, outh_ref, negh_ref, xbar_ref, summ_ref,
    sa, sb, sc, sd, se, sf, sg, adj_s,
    sem_a, sem_b, sem_c, sem_d, sem_e, sem_f, sem_g, sem_adj,
    *, v,
):
    f32 = jnp.float32
    bf16 = jnp.bfloat16
    s = _SIGMA
    n = x_ref.shape[0]

    def start(hbm, buf, sem):
        pltpu.make_async_copy(hbm, buf, sem).start()

    def wait(hbm, buf, sem):
        pltpu.make_async_copy(hbm, buf, sem).wait()

    start(e1w_hbm, sa, sem_a)
    start(e2w_hbm, sb, sem_b)
    start(e3w_hbm, sc, sem_c)
    start(zw_hbm, sd, sem_d)
    start(adj_hbm, adj_s, sem_adj)
    start(d1w_hbm, se, sem_e)
    start(d2w_hbm, sf, sem_f)
    start(xbw_hbm, sg, sem_g)

    xb = x_ref[...].astype(bf16)

    def lin(xv, w_ref, b_ref, relu):
        y = jnp.dot(xv.astype(bf16), w_ref[...].astype(bf16),
                    preferred_element_type=f32)
        y = y + b_ref[...]
        return jnp.maximum(y, 0.0) if relu else y

    wait(e1w_hbm, sa, sem_a)
    t1 = lin(xb, sa, e1b_ref, True)
    start(g1w_hbm, sa, sem_a)

    wait(e2w_hbm, sb, sem_b)
    t2 = lin(t1, sb, e2b_ref, True)
    start(g2w_hbm, sb, sem_b)

    wait(e3w_hbm, sc, sem_c)
    t3 = lin(t2, sc, e3b_ref, True)
    start(g3w_hbm, sc, sem_c)

    wait(zw_hbm, sd, sem_d)
    z = lin(t3, sd, zb_ref, False)
    start(g4w_hbm, sd, sem_d)

    lane = jax.lax.broadcasted_iota(jnp.int32, (n, n), 1)
    pmat = (lane == permi_ref[...]).astype(bf16)
    tcat = jnp.concatenate(
        [t1.astype(bf16), t2.astype(bf16), t3.astype(bf16)], axis=1)
    ncat = jnp.dot(pmat, tcat, preferred_element_type=f32).astype(bf16)
    c1 = t1.shape[1]
    c2 = c1 + t2.shape[1]
    nt1 = ncat[:, :c1]
    nt2 = ncat[:, c1:c2]
    nt3 = ncat[:, c2:]

    wait(adj_hbm, adj_s, sem_adj)
    adjb = adj_s[...].astype(bf16)

    def agg(sup, relu):
        y = jnp.dot(adjb, sup.astype(bf16), preferred_element_type=f32)
        return jnp.maximum(y, 0.0) if relu else y

    def mm(xv, w_ref):
        return jnp.dot(xv.astype(bf16), w_ref[...].astype(bf16),
                       preferred_element_type=f32)

    wait(g1w_hbm, sa, sem_a)
    h1 = agg(mm(xb, sa), True)

    wait(g2w_hbm, sb, sem_b)
    e2_ = sb.shape[1]
    mix2 = jnp.concatenate([(1.0 - s) * h1 + s * t1,
                            (1.0 - s) * h1 + s * nt1], axis=0)
    sup2 = mm(mix2, sb)
    start(d3w_hbm, sb, sem_b)
    a2 = agg(jnp.concatenate([sup2[:n], sup2[n:]], axis=1), True)
    h2, nh2 = a2[:, :e2_], a2[:, e2_:]

    wait(g3w_hbm, sc, sem_c)
    e3_ = sc.shape[1]
    mix3 = jnp.concatenate([(1.0 - s) * h2 + s * t2,
                            (1.0 - s) * nh2 + s * nt2], axis=0)
    sup3 = mm(mix3, sc)
    a3 = agg(jnp.concatenate([sup3[:n], sup3[n:]], axis=1), True)
    h3, nh3 = a3[:, :e3_], a3[:, e3_:]

    out_h = (1.0 - s) * h3 + s * t3
    neg_h = (1.0 - s) * nh3 + s * nt3

    wait(g4w_hbm, sd, sem_d)
    h4 = agg(mm(out_h, sd), True)
    h5 = agg(jnp.dot(((1.0 - s) * h4 + s * z).astype(bf16),
                     g5w_ref[...].astype(bf16),
                     preferred_element_type=f32), False)

    m = jnp.max(h5, axis=1, keepdims=True)
    e = jnp.exp(h5 - m)
    pred = e * pl.reciprocal(jnp.sum(e, axis=1, keepdims=True), approx=True)

    mean = jnp.mean(out_h, axis=0, keepdims=True)
    en = jnp.exp(-jnp.abs(mean))
    r = pl.reciprocal(1.0 + en, approx=True)
    summ_ref[...] = jnp.where(mean >= 0.0, r, en * r)

    c = clu_ref[...]
    z2 = jnp.sum(z * z, axis=1, keepdims=True)
    c2 = jnp.sum(c * c, axis=1, keepdims=True).T
    zc = jnp.dot(z, c.T, preferred_element_type=f32)
    dist = jnp.maximum(z2 + c2 - 2.0 * zc, 0.0)
    q = pl.reciprocal(1.0 + dist * (1.0 / v), approx=True)
    if v != 1.0:
        q = q ** ((v + 1.0) / 2.0)
    q = q * pl.reciprocal(jnp.sum(q, axis=1, keepdims=True), approx=True)

    h_ref[...] = h5
    pred_ref[...] = pred
    q_ref[...] = q
    z_ref[...] = z
    outh_ref[...] = out_h
    negh_ref[...] = neg_h

    wait(d1w_hbm, se, sem_e)
    d1 = lin(z, se, d1b_ref, True)
    wait(d2w_hbm, sf, sem_f)
    d2 = lin(d1, sf, d2b_ref, True)
    wait(d3w_hbm, sb, sem_b)
    d3 = lin(d2, sb, d3b_ref, True)
    wait(xbw_hbm, sg, sem_g)
    xbar_ref[...] = lin(d3, sg, xbb_ref, False)


def kernel(enc1_w, enc1_b, enc2_w, enc2_b, enc3_w, enc3_b, z_w, z_b,
           dec1_w, dec1_b, dec2_w, dec2_b, dec3_w, dec3_b, xbar_w, xbar_b,
           gnn1_w, gnn2_w, gnn3_w, gnn4_w, gnn5_w, cluster_layer, disc_weight,
           x, adj, perm):
    n, n_input = x.shape
    n_enc_3 = enc3_w.shape[1]
    n_z = z_w.shape[1]
    n_clusters = gnn5_w.shape[1]
    v = 1.0

    f32 = jnp.float32

    def b2(b):
        return b.reshape(1, -1)

    vmem_ops = [
        x,
        perm.astype(jnp.int32).reshape(n, 1),
        b2(enc1_b), b2(enc2_b), b2(enc3_b), b2(z_b),
        b2(dec1_b), b2(dec2_b), b2(dec3_b), b2(xbar_b),
        gnn5_w, cluster_layer,
    ]
    hbm_ops = [
        adj, enc1_w, enc2_w, enc3_w, z_w,
        dec1_w, dec2_w, dec3_w, xbar_w,
        gnn1_w, gnn2_w, gnn3_w, gnn4_w,
    ]

    in_specs = [_spec2(op.shape) for op in vmem_ops]
    in_specs += [pl.BlockSpec(memory_space=pl.ANY) for _ in hbm_ops]

    out_shapes = (
        jax.ShapeDtypeStruct((n, n_clusters), f32),
        jax.ShapeDtypeStruct((n, n_clusters), f32),
        jax.ShapeDtypeStruct((n, n_clusters), f32),
        jax.ShapeDtypeStruct((n, n_z), f32),
        jax.ShapeDtypeStruct((n, n_enc_3), f32),
        jax.ShapeDtypeStruct((n, n_enc_3), f32),
        jax.ShapeDtypeStruct((n, n_input), f32),
        jax.ShapeDtypeStruct((1, n_enc_3), f32),
    )
    out_specs = [_spec2(sh.shape) for sh in out_shapes]

    scratch_shapes = [
        pltpu.VMEM(enc1_w.shape, f32),
        pltpu.VMEM(enc2_w.shape, f32),
        pltpu.VMEM(enc3_w.shape, f32),
        pltpu.VMEM(z_w.shape, f32),
        pltpu.VMEM(dec1_w.shape, f32),
        pltpu.VMEM(dec2_w.shape, f32),
        pltpu.VMEM(xbar_w.shape, f32),
        pltpu.VMEM(adj.shape, f32),
    ] + [pltpu.SemaphoreType.DMA(())] * 8

    outs = pl.pallas_call(
        functools.partial(_fused_kernel, v=v),
        grid=(1,),
        in_specs=in_specs,
        out_specs=out_specs,
        out_shape=out_shapes,
        scratch_shapes=scratch_shapes,
        compiler_params=pltpu.CompilerParams(
            dimension_semantics=("arbitrary",),
            vmem_limit_bytes=64 * 1024 * 1024,
        ),
    )(*vmem_ops, *hbm_ops)

    h, pred, q, z, out_h, neg_h, x_bar, summary = outs
    return h, out_h, neg_h, summary.reshape(-1), x_bar, q, pred, z


# speedup vs baseline: 1.8947x; 1.0804x over previous
import functools

import jax
import jax.numpy as jnp
from jax.experimental import pallas as pl
from jax.experimental.pallas import tpu as pltpu

_SIGMA = 0.5


def _spec2(shape):
    return pl.BlockSpec(shape, lambda i: (0, 0))


def _fused_kernel(
    permi_ref,
    e1b_ref, e2b_ref, e3b_ref, zb_ref,
    d1b_ref, d2b_ref, d3b_ref, xbb_ref,
    g5w_ref, clu_ref,
    x_hbm, adj_hbm, e1w_hbm, e2w_hbm, e3w_hbm, zw_hbm,
    d1w_hbm, d2w_hbm, d3w_hbm, xbw_hbm,
    g1w_hbm, g2w_hbm, g3w_hbm, g4w_hbm,
    h_hbm, pred_hbm, q_hbm, z_hbm, outh_hbm, negh_hbm, xbar_hbm, summ_hbm,
    sa, sb, sc, sd, se, sf, sg, adj_s, x_s,
    ho_s, po_s, qo_s, zo_s, oh_s, nh_s, xb_s, su_s,
    sem_a, sem_b, sem_c, sem_d, sem_e, sem_f, sem_g, sem_adj, sem_x,
    sem_o1, sem_o2,
    *, v,
):
    f32 = jnp.float32
    bf16 = jnp.bfloat16
    s = _SIGMA
    n = x_s.shape[0]

    def start(src, dst, sem):
        pltpu.make_async_copy(src, dst, sem).start()

    def wait(src, dst, sem):
        pltpu.make_async_copy(src, dst, sem).wait()

    start(e1w_hbm, sa, sem_a)
    start(x_hbm, x_s, sem_x)
    start(e2w_hbm, sb, sem_b)
    start(e3w_hbm, sc, sem_c)
    start(zw_hbm, sd, sem_d)
    start(adj_hbm, adj_s, sem_adj)
    start(d1w_hbm, se, sem_e)
    start(d2w_hbm, sf, sem_f)
    start(xbw_hbm, sg, sem_g)

    def lin(xv, w_ref, b_ref, relu):
        y = jnp.dot(xv.astype(bf16), w_ref[...].astype(bf16),
                    preferred_element_type=f32)
        y = y + b_ref[...]
        return jnp.maximum(y, 0.0) if relu else y

    wait(x_hbm, x_s, sem_x)
    xb = x_s[...].astype(bf16)

    wait(e1w_hbm, sa, sem_a)
    t1 = lin(xb, sa, e1b_ref, True)
    start(g1w_hbm, sa, sem_a)

    wait(e2w_hbm, sb, sem_b)
    t2 = lin(t1, sb, e2b_ref, True)
    start(g2w_hbm, sb, sem_b)

    wait(e3w_hbm, sc, sem_c)
    t3 = lin(t2, sc, e3b_ref, True)
    start(g3w_hbm, sc, sem_c)

    wait(zw_hbm, sd, sem_d)
    z = lin(t3, sd, zb_ref, False)
    start(g4w_hbm, sd, sem_d)

    zo_s[...] = z
    start(zo_s, z_hbm, sem_o1)

    lane = jax.lax.broadcasted_iota(jnp.int32, (n, n), 1)
    pmat = (lane == permi_ref[...]).astype(bf16)
    tcat = jnp.concatenate(
        [t1.astype(bf16), t2.astype(bf16), t3.astype(bf16)], axis=1)
    ncat = jnp.dot(pmat, tcat, preferred_element_type=f32).astype(bf16)
    c1 = t1.shape[1]
    c2 = c1 + t2.shape[1]
    nt1 = ncat[:, :c1]
    nt2 = ncat[:, c1:c2]
    nt3 = ncat[:, c2:]

    wait(adj_hbm, adj_s, sem_adj)
    adjb = adj_s[...].astype(bf16)

    def agg(sup, relu):
        y = jnp.dot(adjb, sup.astype(bf16), preferred_element_type=f32)
        return jnp.maximum(y, 0.0) if relu else y

    def mm(xv, w_ref):
        return jnp.dot(xv.astype(bf16), w_ref[...].astype(bf16),
                       preferred_element_type=f32)

    wait(g1w_hbm, sa, sem_a)
    h1 = agg(mm(xb, sa), True)

    wait(g2w_hbm, sb, sem_b)
    e2_ = sb.shape[1]
    mix2 = jnp.concatenate([(1.0 - s) * h1 + s * t1,
                            (1.0 - s) * h1 + s * nt1], axis=0)
    sup2 = mm(mix2, sb)
    start(d3w_hbm, sb, sem_b)
    a2 = agg(jnp.concatenate([sup2[:n], sup2[n:]], axis=1), True)
    h2, nh2 = a2[:, :e2_], a2[:, e2_:]

    wait(g3w_hbm, sc, sem_c)
    e3_ = sc.shape[1]
    mix3 = jnp.concatenate([(1.0 - s) * h2 + s * t2,
                            (1.0 - s) * nh2 + s * nt2], axis=0)
    sup3 = mm(mix3, sc)
    a3 = agg(jnp.concatenate([sup3[:n], sup3[n:]], axis=1), True)
    h3, nh3 = a3[:, :e3_], a3[:, e3_:]

    out_h = (1.0 - s) * h3 + s * t3
    neg_h = (1.0 - s) * nh3 + s * nt3

    oh_s[...] = out_h
    start(oh_s, outh_hbm, sem_o1)
    nh_s[...] = neg_h
    start(nh_s, negh_hbm, sem_o1)

    wait(g4w_hbm, sd, sem_d)
    h4 = agg(mm(out_h, sd), True)
    h5 = agg(jnp.dot(((1.0 - s) * h4 + s * z).astype(bf16),
                     g5w_ref[...].astype(bf16),
                     preferred_element_type=f32), False)

    m = jnp.max(h5, axis=1, keepdims=True)
    e = jnp.exp(h5 - m)
    pred = e * pl.reciprocal(jnp.sum(e, axis=1, keepdims=True), approx=True)

    mean = jnp.mean(out_h, axis=0, keepdims=True)
    en = jnp.exp(-jnp.abs(mean))
    r = pl.reciprocal(1.0 + en, approx=True)
    su_s[...] = jnp.where(mean >= 0.0, r, en * r)

    c = clu_ref[...]
    z2 = jnp.sum(z * z, axis=1, keepdims=True)
    c2 = jnp.sum(c * c, axis=1, keepdims=True).T
    zc = jnp.dot(z, c.T, preferred_element_type=f32)
    dist = jnp.maximum(z2 + c2 - 2.0 * zc, 0.0)
    q = pl.reciprocal(1.0 + dist * (1.0 / v), approx=True)
    if v != 1.0:
        q = q ** ((v + 1.0) / 2.0)
    q = q * pl.reciprocal(jnp.sum(q, axis=1, keepdims=True), approx=True)

    ho_s[...] = h5
    po_s[...] = pred
    qo_s[...] = q
    start(ho_s, h_hbm, sem_o1)
    start(po_s, pred_hbm, sem_o1)
    start(qo_s, q_hbm, sem_o1)
    start(su_s, summ_hbm, sem_o1)

    wait(d1w_hbm, se, sem_e)
    d1 = lin(z, se, d1b_ref, True)
    wait(d2w_hbm, sf, sem_f)
    d2 = lin(d1, sf, d2b_ref, True)
    wait(d3w_hbm, sb, sem_b)
    d3 = lin(d2, sb, d3b_ref, True)
    wait(xbw_hbm, sg, sem_g)
    xb_s[...] = lin(d3, sg, xbb_ref, False)
    start(xb_s, xbar_hbm, sem_o2)

    wait(zo_s, z_hbm, sem_o1)
    wait(oh_s, outh_hbm, sem_o1)
    wait(nh_s, negh_hbm, sem_o1)
    wait(ho_s, h_hbm, sem_o1)
    wait(po_s, pred_hbm, sem_o1)
    wait(qo_s, q_hbm, sem_o1)
    wait(su_s, summ_hbm, sem_o1)
    wait(xb_s, xbar_hbm, sem_o2)


def kernel(enc1_w, enc1_b, enc2_w, enc2_b, enc3_w, enc3_b, z_w, z_b,
           dec1_w, dec1_b, dec2_w, dec2_b, dec3_w, dec3_b, xbar_w, xbar_b,
           gnn1_w, gnn2_w, gnn3_w, gnn4_w, gnn5_w, cluster_layer, disc_weight,
           x, adj, perm):
    n, n_input = x.shape
    n_enc_3 = enc3_w.shape[1]
    n_z = z_w.shape[1]
    n_clusters = gnn5_w.shape[1]
    v = 1.0

    f32 = jnp.float32

    def b2(b):
        return b.reshape(1, -1)

    vmem_ops = [
        perm.astype(jnp.int32).reshape(n, 1),
        b2(enc1_b), b2(enc2_b), b2(enc3_b), b2(z_b),
        b2(dec1_b), b2(dec2_b), b2(dec3_b), b2(xbar_b),
        gnn5_w, cluster_layer,
    ]
    hbm_ops = [
        x, adj, enc1_w, enc2_w, enc3_w, z_w,
        dec1_w, dec2_w, dec3_w, xbar_w,
        gnn1_w, gnn2_w, gnn3_w, gnn4_w,
    ]

    in_specs = [_spec2(op.shape) for op in vmem_ops]
    in_specs += [pl.BlockSpec(memory_space=pl.ANY) for _ in hbm_ops]

    out_shapes = (
        jax.ShapeDtypeStruct((n, n_clusters), f32),
        jax.ShapeDtypeStruct((n, n_clusters), f32),
        jax.ShapeDtypeStruct((n, n_clusters), f32),
        jax.ShapeDtypeStruct((n, n_z), f32),
        jax.ShapeDtypeStruct((n, n_enc_3), f32),
        jax.ShapeDtypeStruct((n, n_enc_3), f32),
        jax.ShapeDtypeStruct((n, n_input), f32),
        jax.ShapeDtypeStruct((1, n_enc_3), f32),
    )
    out_specs = [pl.BlockSpec(memory_space=pl.ANY) for _ in out_shapes]

    scratch_shapes = [
        pltpu.VMEM(enc1_w.shape, f32),
        pltpu.VMEM(enc2_w.shape, f32),
        pltpu.VMEM(enc3_w.shape, f32),
        pltpu.VMEM(z_w.shape, f32),
        pltpu.VMEM(dec1_w.shape, f32),
        pltpu.VMEM(dec2_w.shape, f32),
        pltpu.VMEM(xbar_w.shape, f32),
        pltpu.VMEM(adj.shape, f32),
        pltpu.VMEM(x.shape, f32),
    ] + [pltpu.VMEM(sh.shape, f32) for sh in out_shapes] \
      + [pltpu.SemaphoreType.DMA(())] * 11

    outs = pl.pallas_call(
        functools.partial(_fused_kernel, v=v),
        grid=(1,),
        in_specs=in_specs,
        out_specs=out_specs,
        out_shape=out_shapes,
        scratch_shapes=scratch_shapes,
        compiler_params=pltpu.CompilerParams(
            dimension_semantics=("arbitrary",),
            vmem_limit_bytes=64 * 1024 * 1024,
        ),
    )(*vmem_ops, *hbm_ops)

    h, pred, q, z, out_h, neg_h, x_bar, summary = outs
    return h, out_h, neg_h, summary.reshape(-1), x_bar, q, pred, z
